# baseline (speedup 1.0000x reference)
import functools

import jax
import jax.numpy as jnp
from jax import lax
from jax.experimental import pallas as pl
from jax.experimental.pallas import tpu as pltpu
from jax.experimental.pallas import tpu_sc as plsc

FEATS = 50
LANES = 16
TABLE = 1024
NC, NS = 2, 16


@functools.cache
def _build(B: int, L: int):
    nw = NC * NS
    rows_per_w = B // nw
    assert rows_per_w % 2 == 0, "double buffering wants an even row count"
    row_words = L * FEATS
    nvec = (L + LANES - 1) // LANES
    in_words = rows_per_w * L

    mesh = plsc.VectorSubcoreMesh(core_axis_name="c", subcore_axis_name="s")

    @functools.partial(
        pl.kernel,
        out_type=(
            jax.ShapeDtypeStruct((B, row_words), jnp.float32),
            jax.ShapeDtypeStruct((B, row_words), jnp.float32),
        ),
        mesh=mesh,
        compiler_params=pltpu.CompilerParams(needs_layout_passes=False),
        scratch_types=[
            pltpu.VMEM((in_words + LANES,), jnp.int32),
            pltpu.VMEM((in_words + LANES,), jnp.int32),
            pltpu.VMEM((TABLE,), jnp.int32),
            pltpu.VMEM((TABLE,), jnp.int32),
            pltpu.VMEM((row_words,), jnp.float32),
            pltpu.VMEM((row_words,), jnp.float32),
            pltpu.VMEM((row_words,), jnp.float32),
            pltpu.VMEM((row_words,), jnp.float32),
            pltpu.SemaphoreType.DMA,
            pltpu.SemaphoreType.DMA,
            pltpu.SemaphoreType.DMA,
            pltpu.SemaphoreType.DMA,
        ],
    )
    def sc_kernel(src_hbm, dst_hbm, out_s, out_d,
                  src_v, dst_v, tab_s, tab_d,
                  obuf_s0, obuf_s1, obuf_d0, obuf_d1,
                  sem_s0, sem_s1, sem_d0, sem_d1):
        wid = lax.axis_index("s") * NC + lax.axis_index("c")
        base_row = wid * rows_per_w

        pltpu.sync_copy(src_hbm.at[pl.ds(base_row * L, in_words)],
                        src_v.at[pl.ds(0, in_words)])
        pltpu.sync_copy(dst_hbm.at[pl.ds(base_row * L, in_words)],
                        dst_v.at[pl.ds(0, in_words)])

        iota = lax.iota(jnp.int32, LANES)
        zero_f = jnp.zeros((LANES,), jnp.float32)
        neg_i = jnp.full((LANES,), -1, jnp.int32)
        zero_i = jnp.zeros((LANES,), jnp.int32)

        def init_body(i, c):
            o = i * LANES
            obuf_s0[pl.ds(o, LANES)] = zero_f
            obuf_s1[pl.ds(o, LANES)] = zero_f
            obuf_d0[pl.ds(o, LANES)] = zero_f
            obuf_d1[pl.ds(o, LANES)] = zero_f

            @pl.when(i < TABLE // LANES)
            def _():
                tab_s[pl.ds(o, LANES)] = neg_i
                tab_d[pl.ds(o, LANES)] = neg_i

            return c

        lax.fori_loop(0, row_words // LANES, init_body, 0)
        src_v[pl.ds(in_words, LANES)] = zero_i
        dst_v[pl.ds(in_words, LANES)] = zero_i

        def process_row(r, obs, obd):
            row_off = r * L
            stamp = zero_i + r

            def stamp_body(i, c):
                off = row_off + i * LANES
                m = (i * LANES + iota) < L
                sv = src_v[pl.ds(off, LANES)] & (TABLE - 1)
                dv = dst_v[pl.ds(off, LANES)] & (TABLE - 1)
                plsc.store_scatter(tab_s, [sv], stamp, mask=m)
                plsc.store_scatter(tab_d, [dv], stamp, mask=m)
                return c

            lax.fori_loop(0, nvec, stamp_body, 0)

            def probe_body(i, c):
                off = row_off + i * LANES
                jv = i * LANES + iota
                m = jv < L
                sv = src_v[pl.ds(off, LANES)] & (TABLE - 1)
                dv = dst_v[pl.ds(off, LANES)] & (TABLE - 1)
                gs = plsc.load_gather(tab_d, [sv])
                gd = plsc.load_gather(tab_s, [dv])
                vs = jnp.where(gs == stamp, 1.0, 0.0).astype(jnp.float32)
                vd = jnp.where(gd == stamp, 1.0, 0.0).astype(jnp.float32)
                oidx = jv * FEATS
                plsc.store_scatter(obs, [oidx], vs, mask=m)
                plsc.store_scatter(obd, [oidx], vd, mask=m)
                return c

            lax.fori_loop(0, nvec, probe_body, 0)

        bufs = ((obuf_s0, obuf_d0, sem_s0, sem_d0),
                (obuf_s1, obuf_d1, sem_s1, sem_d1))

        def pair_body(k, c):
            for p, (obs, obd, ss, sd) in enumerate(bufs):
                r = 2 * k + p
                grow = base_row + r

                @pl.when(k > 0)
                def _():
                    pltpu.make_async_copy(obs, out_s.at[grow - 2], ss).wait()
                    pltpu.make_async_copy(obd, out_d.at[grow - 2], sd).wait()

                process_row(r, obs, obd)
                pltpu.make_async_copy(obs, out_s.at[grow], ss).start()
                pltpu.make_async_copy(obd, out_d.at[grow], sd).start()
            return c

        lax.fori_loop(0, rows_per_w // 2, pair_body, 0)

        tail = base_row + rows_per_w
        pltpu.make_async_copy(obuf_s0, out_s.at[tail - 2], sem_s0).wait()
        pltpu.make_async_copy(obuf_d0, out_d.at[tail - 2], sem_d0).wait()
        pltpu.make_async_copy(obuf_s1, out_s.at[tail - 1], sem_s1).wait()
        pltpu.make_async_copy(obuf_d1, out_d.at[tail - 1], sem_d1).wait()

    return sc_kernel


def kernel(src_padded_nodes_neighbor_ids, dst_padded_nodes_neighbor_ids):
    src = src_padded_nodes_neighbor_ids
    dst = dst_padded_nodes_neighbor_ids
    B, L = src.shape
    sc = _build(B, L)
    out_s, out_d = sc(src.reshape(-1).astype(jnp.int32),
                      dst.reshape(-1).astype(jnp.int32))
    return (out_s.reshape(B, L, FEATS), out_d.reshape(B, L, FEATS))


# speedup vs baseline: 4.5297x; 4.5297x over previous
import functools

import jax
import jax.numpy as jnp
from jax import lax
from jax.experimental import pallas as pl
from jax.experimental.pallas import tpu as pltpu
from jax.experimental.pallas import tpu_sc as plsc

FEATS = 50
LANES = 16
TABLE = 1024
NC, NS = 2, 16


@functools.cache
def _build(B: int, L: int):
    nw = NC * NS
    ltiles = L // 8
    btiles = B // 128
    gsz = nw // btiles
    nr0 = ltiles - (gsz - 1) * (ltiles // gsz)
    nrk = ltiles // gsz
    grp_words = 128 * L
    zrows = 5
    cpp = ltiles // zrows
    nzchunks = (FEATS - 1) * cpp

    mesh = plsc.VectorSubcoreMesh(core_axis_name="c", subcore_axis_name="s")

    @functools.partial(
        pl.kernel,
        out_type=(
            jax.ShapeDtypeStruct((FEATS, ltiles, btiles, 8, 128), jnp.float32),
            jax.ShapeDtypeStruct((FEATS, ltiles, btiles, 8, 128), jnp.float32),
        ),
        mesh=mesh,
        compiler_params=pltpu.CompilerParams(needs_layout_passes=False),
        scratch_types=[
            pltpu.VMEM((grp_words + LANES,), jnp.int32),
            pltpu.VMEM((grp_words + LANES,), jnp.int32),
            pltpu.VMEM((TABLE,), jnp.int32),
            pltpu.VMEM((TABLE,), jnp.int32),
            pltpu.VMEM((nr0, 8, 128), jnp.float32),
            pltpu.VMEM((nr0, 8, 128), jnp.float32),
            pltpu.VMEM((zrows, btiles, 8, 128), jnp.float32),
            pltpu.SemaphoreType.DMA,
            pltpu.SemaphoreType.DMA,
        ],
    )
    def sc_kernel(src_hbm, dst_hbm, out_s, out_d,
                  src_v, dst_v, tab_s, tab_d,
                  mbuf_s, mbuf_d, zbuf, sem_z, sem_m):
        wid = lax.axis_index("s") * NC + lax.axis_index("c")
        tc = wid // gsz
        sub = wid % gsz
        tr0 = jnp.where(sub == 0, 0, nr0 + (sub - 1) * nrk)
        nlim = jnp.where(sub == 0, nr0 * 8, nrk * 8)
        l0 = tr0 * 8

        pltpu.sync_copy(src_hbm.at[pl.ds(tc * grp_words, grp_words)],
                        src_v.at[pl.ds(0, grp_words)])
        pltpu.sync_copy(dst_hbm.at[pl.ds(tc * grp_words, grp_words)],
                        dst_v.at[pl.ds(0, grp_words)])

        iota = lax.iota(jnp.int32, LANES)
        zero_f = jnp.zeros((LANES,), jnp.float32)
        neg_i = jnp.full((LANES,), -1, jnp.int32)
        zero_i = jnp.zeros((LANES,), jnp.int32)

        def zinit_body(t, c):
            c0 = t >> 6
            a = (t >> 3) & 7
            b = t & 7

            def zrow(j, cc):
                zbuf[c0, a, b, pl.ds(j * LANES, LANES)] = zero_f
                return cc

            lax.fori_loop(0, 128 // LANES, zrow, 0)
            return c

        lax.fori_loop(0, zrows * btiles * 8, zinit_body, 0)

        def tinit_body(i, c):
            tab_s[pl.ds(i * LANES, LANES)] = neg_i
            tab_d[pl.ds(i * LANES, LANES)] = neg_i
            return c

        lax.fori_loop(0, TABLE // LANES, tinit_body, 0)
        src_v[pl.ds(grp_words, LANES)] = zero_i
        dst_v[pl.ds(grp_words, LANES)] = zero_i

        zlo = (nzchunks * wid) // nw
        zhi = (nzchunks * (wid + 1)) // nw

        def zfire_body(idx, c):
            ch = 1 + idx // cpp
            zr0 = (idx % cpp) * zrows
            pltpu.make_async_copy(zbuf, out_s.at[ch, pl.ds(zr0, zrows)],
                                  sem_z).start()
            pltpu.make_async_copy(zbuf, out_d.at[ch, pl.ds(zr0, zrows)],
                                  sem_z).start()
            return c

        lax.fori_loop(zlo, zhi, zfire_body, 0)

        nvec = (L + LANES - 1) // LANES
        i0 = iota >> 3
        i1 = iota & 7

        def per_batch(bb, c):
            row_off = bb * L
            stamp = zero_i + bb

            def stamp_body(i, cc):
                off = row_off + i * LANES
                m = (i * LANES + iota) < L
                sv = src_v[pl.ds(off, LANES)] & (TABLE - 1)
                dv = dst_v[pl.ds(off, LANES)] & (TABLE - 1)
                plsc.store_scatter(tab_s, [sv], stamp, mask=m)
                plsc.store_scatter(tab_d, [dv], stamp, mask=m)
                return cc

            lax.fori_loop(0, nvec, stamp_body, 0)

            for i in range((nr0 * 8 + LANES - 1) // LANES):
                lrel = i * LANES + iota
                m = lrel < nlim
                off = row_off + l0 + i * LANES
                sv = src_v[pl.ds(off, LANES)] & (TABLE - 1)
                dv = dst_v[pl.ds(off, LANES)] & (TABLE - 1)
                gs = plsc.load_gather(tab_d, [sv])
                gd = plsc.load_gather(tab_s, [dv])
                vs = jnp.where(gs == stamp, 1.0, 0.0).astype(jnp.float32)
                vd = jnp.where(gd == stamp, 1.0, 0.0).astype(jnp.float32)
                itr = 2 * i + i0
                plsc.store_scatter(mbuf_s, [itr, i1, stamp], vs, mask=m)
                plsc.store_scatter(mbuf_d, [itr, i1, stamp], vd, mask=m)
            return c

        lax.fori_loop(0, 128, per_batch, 0)

        @pl.when(sub == 0)
        def _():
            pltpu.make_async_copy(
                mbuf_s, out_s.at[0, pl.ds(0, nr0), tc], sem_m).start()
            pltpu.make_async_copy(
                mbuf_d, out_d.at[0, pl.ds(0, nr0), tc], sem_m).start()

        @pl.when(sub != 0)
        def _():
            pltpu.make_async_copy(
                mbuf_s.at[pl.ds(0, nrk)], out_s.at[0, pl.ds(tr0, nrk), tc],
                sem_m).start()
            pltpu.make_async_copy(
                mbuf_d.at[pl.ds(0, nrk)], out_d.at[0, pl.ds(tr0, nrk), tc],
                sem_m).start()

        def zdrain_body(idx, c):
            pltpu.make_async_copy(zbuf, out_s.at[1, pl.ds(0, zrows)],
                                  sem_z).wait()
            pltpu.make_async_copy(zbuf, out_d.at[1, pl.ds(0, zrows)],
                                  sem_z).wait()
            return c

        lax.fori_loop(zlo, zhi, zdrain_body, 0)

        @pl.when(sub == 0)
        def _():
            pltpu.make_async_copy(
                mbuf_s, out_s.at[0, pl.ds(0, nr0), tc], sem_m).wait()
            pltpu.make_async_copy(
                mbuf_d, out_d.at[0, pl.ds(0, nr0), tc], sem_m).wait()

        @pl.when(sub != 0)
        def _():
            pltpu.make_async_copy(
                mbuf_s.at[pl.ds(0, nrk)], out_s.at[0, pl.ds(tr0, nrk), tc],
                sem_m).wait()
            pltpu.make_async_copy(
                mbuf_d.at[pl.ds(0, nrk)], out_d.at[0, pl.ds(tr0, nrk), tc],
                sem_m).wait()

    return sc_kernel


def kernel(src_padded_nodes_neighbor_ids, dst_padded_nodes_neighbor_ids):
    src = src_padded_nodes_neighbor_ids
    dst = dst_padded_nodes_neighbor_ids
    B, L = src.shape
    sc = _build(B, L)
    o5_s, o5_d = sc(src.reshape(-1).astype(jnp.int32),
                    dst.reshape(-1).astype(jnp.int32))

    def _untile(o5):
        return o5.transpose(2, 4, 1, 3, 0).reshape(B, L, FEATS)

    return (_untile(o5_s), _untile(o5_d))
